# Initial kernel scaffold; baseline (speedup 1.0000x reference)
import functools

import jax
import jax.numpy as jnp
from jax import lax
from jax.experimental import pallas as pl
from jax.experimental.pallas import tpu as pltpu
from jax.experimental.pallas import tpu_sc as plsc

_DIM = 32
_G = 128
_S = 8
_NC = 2
_NS = 16
_NW = _NC * _NS


@functools.partial(jax.jit, static_argnums=(2,))
def _gather_rows(idx2d, table, n_rows):
    rows_per_w = n_rows // _NW
    chunks = rows_per_w // _S
    mesh = plsc.VectorSubcoreMesh(core_axis_name="c", subcore_axis_name="s")

    @functools.partial(
        pl.kernel,
        mesh=mesh,
        out_type=jax.ShapeDtypeStruct((n_rows * _G, _DIM), jnp.float32),
        scratch_types=[
            pltpu.VMEM((_S, _G), jnp.int32),
            pltpu.VMEM((_S * _G, _DIM), jnp.float32),
            pltpu.SemaphoreType.DMA,
        ],
    )
    def gather(idx_hbm, table_hbm, out_hbm, idx_v, rows_v, sem):
        wid = lax.axis_index("s") * _NC + lax.axis_index("c")
        row0 = wid * rows_per_w

        def body(i, carry):
            r = row0 + i * _S
            pltpu.sync_copy(idx_hbm.at[pl.ds(r, _S)], idx_v)
            copies = [
                pltpu.async_copy(
                    table_hbm.at[idx_v.at[j]],
                    rows_v.at[pl.ds(j * _G, _G)],
                    sem,
                )
                for j in range(_S)
            ]
            for c in copies:
                c.wait()
            pltpu.sync_copy(rows_v, out_hbm.at[pl.ds(r * _G, _S * _G)])
            return carry

        lax.fori_loop(0, chunks, body, 0)

    return gather(idx2d, table)


def kernel(indices, table):
    b, l = indices.shape
    n = b * l
    idx2d = indices.astype(jnp.int32).reshape(n // _G, _G)
    out = _gather_rows(idx2d, table, n // _G)
    return out.reshape(b, l, _DIM)


# speedup vs baseline: 4.8065x; 4.8065x over previous
import functools

import jax
import jax.numpy as jnp
from jax import lax
from jax.experimental import pallas as pl
from jax.experimental.pallas import tpu as pltpu
from jax.experimental.pallas import tpu_sc as plsc

_DIM = 32
_G = 128
_S = 8
_NC = 2
_NS = 16
_NW = _NC * _NS


@functools.partial(jax.jit, static_argnums=(2,))
def _gather_rows(idx2d, table, n_rows):
    rows_per_w = n_rows // _NW
    chunks = rows_per_w // _S
    mesh = plsc.VectorSubcoreMesh(core_axis_name="c", subcore_axis_name="s")

    @functools.partial(
        pl.kernel,
        mesh=mesh,
        out_type=jax.ShapeDtypeStruct((n_rows * _G, _DIM), jnp.float32),
        scratch_types=[
            pltpu.VMEM((_S, _G), jnp.int32),
            pltpu.VMEM((_S * _G, _DIM), jnp.float32),
            pltpu.SemaphoreType.DMA,
        ],
        compiler_params=pltpu.CompilerParams(use_tc_tiling_on_sc=False),
    )
    def gather(idx_hbm, table_hbm, out_hbm, idx_v, rows_v, sem):
        wid = lax.axis_index("s") * _NC + lax.axis_index("c")
        row0 = wid * rows_per_w

        def body(i, carry):
            r = row0 + i * _S
            pltpu.sync_copy(idx_hbm.at[pl.ds(r, _S)], idx_v)
            copies = [
                pltpu.async_copy(
                    table_hbm.at[idx_v.at[j]],
                    rows_v.at[pl.ds(j * _G, _G)],
                    sem,
                )
                for j in range(_S)
            ]
            for c in copies:
                c.wait()
            pltpu.sync_copy(rows_v, out_hbm.at[pl.ds(r * _G, _S * _G)])
            return carry

        lax.fori_loop(0, chunks, body, 0)

    return gather(idx2d, table)


def kernel(indices, table):
    b, l = indices.shape
    n = b * l
    idx2d = indices.astype(jnp.int32).reshape(n // _G, _G)
    out = _gather_rows(idx2d, table, n // _G)
    return out.reshape(b, l, _DIM)
